# baseline (speedup 1.0000x reference)
import functools

import jax
import jax.numpy as jnp
from jax import lax
from jax.experimental import pallas as pl
from jax.experimental.pallas import tpu as pltpu
from jax.experimental.pallas import tpu_sc as plsc

BATCH = 16384
D = 32
NW = 32
B_PER_W = BATCH // NW
CHUNK = 128
NCHUNK = B_PER_W // CHUNK


@functools.partial(
    pl.kernel,
    out_type=jax.ShapeDtypeStruct((BATCH, 2 * D), jnp.float32),
    mesh=plsc.VectorSubcoreMesh(core_axis_name="c", subcore_axis_name="s"),
    compiler_params=pltpu.CompilerParams(use_tc_tiling_on_sc=False),
    scratch_types=[
        pltpu.VMEM((B_PER_W,), jnp.int32),
        pltpu.VMEM((B_PER_W,), jnp.int32),
        pltpu.VMEM((B_PER_W, D), jnp.float32),
        pltpu.VMEM((B_PER_W, D), jnp.float32),
        pltpu.SemaphoreType.DMA,
    ],
)
def _lookup_concat(cid_hbm, oid_hbm, user_hbm, org_hbm, out_hbm,
                   cidx_v, oidx_v, urows_v, orows_v, sem):
    wid = lax.axis_index("s") * 2 + lax.axis_index("c")
    base = wid * B_PER_W

    pltpu.sync_copy(cid_hbm.at[pl.ds(base, B_PER_W)], cidx_v)
    pltpu.sync_copy(oid_hbm.at[pl.ds(base, B_PER_W)], oidx_v)

    copies = []
    for j in range(NCHUNK):
        copies.append(pltpu.async_copy(
            user_hbm.at[cidx_v.at[pl.ds(j * CHUNK, CHUNK)]],
            urows_v.at[pl.ds(j * CHUNK, CHUNK)], sem))
    for j in range(NCHUNK):
        copies.append(pltpu.async_copy(
            org_hbm.at[oidx_v.at[pl.ds(j * CHUNK, CHUNK)]],
            orows_v.at[pl.ds(j * CHUNK, CHUNK)], sem))
    for c in copies:
        c.wait()

    pltpu.sync_copy(urows_v, out_hbm.at[pl.ds(base, B_PER_W), pl.ds(0, D)])
    pltpu.sync_copy(orows_v, out_hbm.at[pl.ds(base, B_PER_W), pl.ds(D, D)])


def kernel(clientId, organization, user_table, org_table):
    cid = clientId.astype(jnp.int32)
    oid = organization.astype(jnp.int32)
    return _lookup_concat(cid, oid, user_table, org_table)
